# Initial kernel scaffold; baseline (speedup 1.0000x reference)
import functools

import numpy as np
import jax
import jax.numpy as jnp
from jax import lax
from jax.experimental import pallas as pl
from jax.experimental.pallas import tpu as pltpu
from jax.experimental.pallas import tpu_sc as plsc

N = 10000
D = 128
NPAD = 10240
NCORES = 2
NTILES = 16
NW = NCORES * NTILES
RPT = NPAD // NTILES
C = 128
BM = 1000


def _sc_mesh():
    return plsc.VectorSubcoreMesh(core_axis_name="c", subcore_axis_name="s")


@functools.lru_cache()
def _make_deg(nch):
    @functools.partial(
        pl.kernel,
        mesh=_sc_mesh(),
        out_type=jax.ShapeDtypeStruct((NCORES, NPAD, 1), jnp.float32),
        scratch_types=[
            pltpu.VMEM((nch, C), jnp.int32),
            pltpu.VMEM((C, 1), jnp.float32),
            pltpu.VMEM_SHARED((NPAD, 1), jnp.float32),
        ],
    )
    def deg_kernel(dstp_hbm, ones_hbm, zeros1_hbm, out_hbm, dst_idx, ones_v, acc):
        c = lax.axis_index("c")
        s = lax.axis_index("s")
        wid = c * NTILES + s
        pltpu.sync_copy(dstp_hbm.at[wid], dst_idx)
        pltpu.sync_copy(ones_hbm, ones_v)
        r0 = s * RPT
        pltpu.sync_copy(zeros1_hbm.at[pl.ds(r0, RPT)], acc.at[pl.ds(r0, RPT)])
        plsc.subcore_barrier()

        def step(j, carry):
            pltpu.sync_copy(ones_v, acc.at[dst_idx.at[j]], add=True)
            return carry

        lax.fori_loop(0, nch, step, 0)
        plsc.subcore_barrier()
        pltpu.sync_copy(acc.at[pl.ds(r0, RPT)], out_hbm.at[c, pl.ds(r0, RPT)])

    return deg_kernel


@functools.lru_cache()
def _make_agg(nch):
    @functools.partial(
        pl.kernel,
        mesh=_sc_mesh(),
        out_type=jax.ShapeDtypeStruct((NCORES, NPAD, D), jnp.float32),
        scratch_types=[
            pltpu.VMEM((nch, C), jnp.int32),
            pltpu.VMEM((nch, C), jnp.int32),
            pltpu.VMEM((C, D), jnp.float32),
            pltpu.VMEM((C, D), jnp.float32),
            pltpu.VMEM_SHARED((NPAD, D), jnp.float32),
            pltpu.SemaphoreType.DMA,
            pltpu.SemaphoreType.DMA,
        ],
    )
    def agg_kernel(y_hbm, srcp_hbm, dstp_hbm, zeros_hbm, out_hbm,
                   src_idx, dst_idx, rows0, rows1, acc, sem0, sem1):
        c = lax.axis_index("c")
        s = lax.axis_index("s")
        wid = c * NTILES + s
        pltpu.sync_copy(srcp_hbm.at[wid], src_idx)
        pltpu.sync_copy(dstp_hbm.at[wid], dst_idx)
        r0 = s * RPT
        pltpu.sync_copy(zeros_hbm.at[pl.ds(r0, RPT)], acc.at[pl.ds(r0, RPT)])
        plsc.subcore_barrier()

        pltpu.async_copy(y_hbm.at[src_idx.at[0]], rows0, sem0)
        pltpu.async_copy(y_hbm.at[src_idx.at[1]], rows1, sem1)

        def step(i, carry):
            j = i * 2
            for b, rows, sem in ((0, rows0, sem0), (1, rows1, sem1)):
                jb = j + b
                pltpu.make_async_copy(y_hbm.at[pl.ds(0, C)], rows, sem).wait()
                pltpu.sync_copy(rows, acc.at[dst_idx.at[jb]], add=True)

                @pl.when(jb + 2 < nch)
                def _():
                    pltpu.async_copy(y_hbm.at[src_idx.at[jb + 2]], rows, sem)
            return carry

        lax.fori_loop(0, nch // 2, step, 0)
        plsc.subcore_barrier()
        pltpu.sync_copy(acc.at[pl.ds(r0, RPT)], out_hbm.at[c, pl.ds(r0, RPT)])

    return agg_kernel


def _t1_body(p0, p1, x, w, y, dinv):
    deg = 1.0 + p0[0] + p1[0]
    di = lax.rsqrt(deg)
    y[...] = jnp.dot(x[...], w[...], preferred_element_type=jnp.float32) * di
    dinv[...] = di


def _t1(degp, x, w1):
    grid = (N // BM,)
    return pl.pallas_call(
        _t1_body,
        grid=grid,
        in_specs=[
            pl.BlockSpec((1, BM, 1), lambda i: (0, i, 0)),
            pl.BlockSpec((1, BM, 1), lambda i: (1, i, 0)),
            pl.BlockSpec((BM, D), lambda i: (i, 0)),
            pl.BlockSpec((D, D), lambda i: (0, 0)),
        ],
        out_specs=[
            pl.BlockSpec((BM, D), lambda i: (i, 0)),
            pl.BlockSpec((BM, 1), lambda i: (i, 0)),
        ],
        out_shape=[
            jax.ShapeDtypeStruct((N, D), jnp.float32),
            jax.ShapeDtypeStruct((N, 1), jnp.float32),
        ],
    )(degp, degp, x, w1)


def _t2_body(q0, q1, y, dinv, b, w, out):
    h = dinv[...] * (q0[0] + q1[0] + y[...]) + b[...]
    h = jnp.maximum(h, 0.0)
    out[...] = jnp.dot(h, w[...], preferred_element_type=jnp.float32) * dinv[...]


def _t2(q, y, dinv, b, w):
    grid = (N // BM,)
    return pl.pallas_call(
        _t2_body,
        grid=grid,
        in_specs=[
            pl.BlockSpec((1, BM, D), lambda i: (0, i, 0)),
            pl.BlockSpec((1, BM, D), lambda i: (1, i, 0)),
            pl.BlockSpec((BM, D), lambda i: (i, 0)),
            pl.BlockSpec((BM, 1), lambda i: (i, 0)),
            pl.BlockSpec((1, D), lambda i: (0, 0)),
            pl.BlockSpec((D, D), lambda i: (0, 0)),
        ],
        out_specs=pl.BlockSpec((BM, D), lambda i: (i, 0)),
        out_shape=jax.ShapeDtypeStruct((N, D), jnp.float32),
    )(q, q, y, dinv, b.reshape(1, D), w)


def _t3_body(q0, q1, y, dinv, b, fcw, fcb, out, acc):
    i = pl.program_id(0)

    @pl.when(i == 0)
    def _():
        acc[...] = jnp.zeros_like(acc)

    h = dinv[...] * (q0[0] + q1[0] + y[...]) + b[...]
    h = jnp.maximum(h, 0.0)
    acc[...] += jnp.sum(h, axis=0, keepdims=True)

    @pl.when(i == pl.num_programs(0) - 1)
    def _():
        g = acc[...] * (1.0 / N)
        logits = jnp.dot(g, fcw[...], preferred_element_type=jnp.float32) + fcb[...]
        m = jnp.max(logits, axis=1, keepdims=True)
        t = logits - m
        out[...] = t - jnp.log(jnp.sum(jnp.exp(t), axis=1, keepdims=True))


def _t3(q, y, dinv, b, fcw, fcb):
    grid = (N // BM,)
    return pl.pallas_call(
        _t3_body,
        grid=grid,
        in_specs=[
            pl.BlockSpec((1, BM, D), lambda i: (0, i, 0)),
            pl.BlockSpec((1, BM, D), lambda i: (1, i, 0)),
            pl.BlockSpec((BM, D), lambda i: (i, 0)),
            pl.BlockSpec((BM, 1), lambda i: (i, 0)),
            pl.BlockSpec((1, D), lambda i: (0, 0)),
            pl.BlockSpec((D, D), lambda i: (0, 0)),
            pl.BlockSpec((1, D), lambda i: (0, 0)),
        ],
        out_specs=pl.BlockSpec((1, D), lambda i: (0, 0)),
        out_shape=jax.ShapeDtypeStruct((1, D), jnp.float32),
        scratch_shapes=[pltpu.VMEM((1, D), jnp.float32)],
    )(q, q, y, dinv, b.reshape(1, D), fcw, fcb.reshape(1, D))


def kernel(x, edge_index, W1, b1, W2, b2, fc_W, fc_b):
    src = edge_index[0].astype(jnp.int32)
    dst = edge_index[1].astype(jnp.int32)
    e = src.shape[0]

    nch = -(-e // (NW * C))
    if nch % 2:
        nch += 1
    epad = NW * nch * C
    pad_n = epad - e
    if pad_n:
        pad_src = jnp.asarray((np.arange(pad_n) * 131) % N, jnp.int32)
        pad_dst = jnp.asarray(N + (np.arange(pad_n) % (NPAD - N)), jnp.int32)
        src = jnp.concatenate([src, pad_src])
        dst = jnp.concatenate([dst, pad_dst])
    srcp = src.reshape(NW, nch, C)
    dstp = dst.reshape(NW, nch, C)

    zeros = jnp.zeros((NPAD, D), jnp.float32)
    zeros1 = jnp.zeros((NPAD, 1), jnp.float32)
    ones_c = jnp.ones((C, 1), jnp.float32)

    deg_kernel = _make_deg(nch)
    agg_kernel = _make_agg(nch)

    degp = deg_kernel(dstp, ones_c, zeros1)
    y1, dinv = _t1(degp, x, W1)
    q1 = agg_kernel(y1, srcp, dstp, zeros)
    y2 = _t2(q1, y1, dinv, b1, W2)
    q2 = agg_kernel(y2, srcp, dstp, zeros)
    return _t3(q2, y2, dinv, b2, fc_W, fc_b)


# speedup vs baseline: 31.6401x; 31.6401x over previous
import functools

import numpy as np
import jax
import jax.numpy as jnp
from jax import lax
from jax.experimental import pallas as pl
from jax.experimental.pallas import tpu as pltpu
from jax.experimental.pallas import tpu_sc as plsc

N = 10000
D = 128
DW = 16
NPAD = 10240
NCORES = 2
NTILES = 16
NW = NCORES * NTILES
RPT = NPAD // NTILES
C = 64
RING = 3
C2 = 128
BM = 1000


def _sc_mesh():
    return plsc.VectorSubcoreMesh(
        core_axis_name="c", subcore_axis_name="s",
        num_cores=NCORES, num_subcores=NTILES,
    )


@functools.lru_cache()
def _make_deg(nch2):
    @functools.partial(
        pl.kernel,
        mesh=_sc_mesh(),
        compiler_params=pltpu.CompilerParams(use_tc_tiling_on_sc=False),
        out_type=jax.ShapeDtypeStruct((NCORES, NPAD, DW), jnp.float32),
        scratch_types=[
            pltpu.VMEM((nch2, C2), jnp.int32),
            pltpu.VMEM((C2, DW), jnp.float32),
            pltpu.VMEM_SHARED((NPAD, DW), jnp.float32),
            pltpu.SemaphoreType.DMA((2,)),
        ],
    )
    def deg_kernel(dstp_hbm, ones_hbm, zeros1_hbm, out_hbm, dst_idx, ones_v, acc, sem_o):
        c = lax.axis_index("c")
        s = lax.axis_index("s")
        wid = c * NTILES + s
        pltpu.sync_copy(dstp_hbm.at[wid], dst_idx)
        pltpu.sync_copy(ones_hbm, ones_v)
        r0 = s * RPT
        pltpu.sync_copy(zeros1_hbm.at[pl.ds(r0, RPT)], acc.at[pl.ds(r0, RPT)])
        plsc.subcore_barrier()

        def step(j, carry):
            b = lax.rem(j, 2)

            @pl.when(j >= 2)
            def _():
                pltpu.make_async_copy(ones_hbm, ones_v, sem_o.at[b]).wait()

            pltpu.async_copy(ones_v, acc.at[dst_idx.at[j]], sem_o.at[b],
                             add=True)
            return carry

        lax.fori_loop(0, nch2, step, 0)

        def dwait(k, carry):
            pltpu.make_async_copy(ones_hbm, ones_v, sem_o.at[k]).wait()
            return carry

        lax.fori_loop(0, 2, dwait, 0)
        plsc.subcore_barrier()
        pltpu.sync_copy(acc.at[pl.ds(r0, RPT)], out_hbm.at[c, pl.ds(r0, RPT)])

    return deg_kernel


@functools.lru_cache()
def _make_agg(nch):
    @functools.partial(
        pl.kernel,
        mesh=_sc_mesh(),
        compiler_params=pltpu.CompilerParams(use_tc_tiling_on_sc=False),
        out_type=jax.ShapeDtypeStruct((NCORES, NPAD, D), jnp.float32),
        scratch_types=[
            pltpu.VMEM_SHARED((NPAD, D), jnp.float32),
            pltpu.SemaphoreType.DMA((RING,)),
            pltpu.SemaphoreType.DMA((RING,)),
            pltpu.SemaphoreType.DMA,
        ],
    )
    def agg_kernel(y_hbm, edges_hbm, zeros_hbm, out_hbm, acc, sem_g, sem_s, sem_z):
        c = lax.axis_index("c")
        s = lax.axis_index("s")
        wid = c * NTILES + s
        r0 = s * RPT

        CR = 64

        def inner(idx, rows):
            pltpu.sync_copy(edges_hbm.at[wid], idx)

            def zstep(k, carry):
                off = r0 + k * CR
                pltpu.async_copy(zeros_hbm.at[pl.ds(off, CR)],
                                 acc.at[pl.ds(off, CR)], sem_z)
                return carry

            lax.fori_loop(0, RPT // CR, zstep, 0)

            def zwait(k, carry):
                pltpu.make_async_copy(zeros_hbm.at[pl.ds(r0, CR)],
                                      acc.at[pl.ds(r0, CR)], sem_z).wait()
                return carry

            lax.fori_loop(0, RPT // CR, zwait, 0)
            plsc.subcore_barrier()

            def step(j, carry):
                b = lax.rem(j, RING)

                @pl.when(j >= RING)
                def _():
                    pltpu.make_async_copy(y_hbm.at[pl.ds(0, C)], rows.at[b],
                                          sem_s.at[b]).wait()

                @pl.when(j < nch)
                def _():
                    pltpu.async_copy(y_hbm.at[idx.at[j, 0]], rows.at[b],
                                     sem_g.at[b])

                @pl.when(j >= 1)
                def _():
                    jp = j - 1
                    bp = lax.rem(jp, RING)
                    pltpu.make_async_copy(y_hbm.at[pl.ds(0, C)], rows.at[bp],
                                          sem_g.at[bp]).wait()
                    pltpu.async_copy(rows.at[bp], acc.at[idx.at[jp, 1]],
                                     sem_s.at[bp], add=True)

                return carry

            lax.fori_loop(0, nch + 1, step, 0)

            def dstep(k, carry):
                bd = lax.rem(nch - (RING - 1) + k, RING)
                pltpu.make_async_copy(y_hbm.at[pl.ds(0, C)], rows.at[bd],
                                      sem_s.at[bd]).wait()
                return carry

            lax.fori_loop(0, RING - 1, dstep, 0)
            plsc.subcore_barrier()

            def ostep(k, carry):
                off = r0 + k * CR
                pltpu.async_copy(acc.at[pl.ds(off, CR)],
                                 out_hbm.at[c, pl.ds(off, CR)], sem_z)
                return carry

            lax.fori_loop(0, RPT // CR, ostep, 0)

            def owait(k, carry):
                pltpu.make_async_copy(acc.at[pl.ds(r0, CR)],
                                      out_hbm.at[c, pl.ds(r0, CR)], sem_z).wait()
                return carry

            lax.fori_loop(0, RPT // CR, owait, 0)

        pl.run_scoped(
            inner,
            pltpu.VMEM((nch, 2, C), jnp.int32),
            pltpu.VMEM((RING, C, D), jnp.float32),
        )

    return agg_kernel


def _t1_body(p0, p1, x, w, y, dinv):
    deg = 1.0 + p0[0][:, :1] + p1[0][:, :1]
    di = lax.rsqrt(deg)
    y[...] = jnp.dot(x[...], w[...], preferred_element_type=jnp.float32) * di
    dinv[...] = di


def _t1(degp, x, w1):
    grid = (N // BM,)
    return pl.pallas_call(
        _t1_body,
        grid=grid,
        in_specs=[
            pl.BlockSpec((1, BM, DW), lambda i: (0, i, 0)),
            pl.BlockSpec((1, BM, DW), lambda i: (1, i, 0)),
            pl.BlockSpec((BM, D), lambda i: (i, 0)),
            pl.BlockSpec((D, D), lambda i: (0, 0)),
        ],
        out_specs=[
            pl.BlockSpec((BM, D), lambda i: (i, 0)),
            pl.BlockSpec((BM, 1), lambda i: (i, 0)),
        ],
        out_shape=[
            jax.ShapeDtypeStruct((N, D), jnp.float32),
            jax.ShapeDtypeStruct((N, 1), jnp.float32),
        ],
    )(degp, degp, x, w1)


def _t2_body(q0, q1, y, dinv, b, w, out):
    h = dinv[...] * (q0[0] + q1[0] + y[...]) + b[...]
    h = jnp.maximum(h, 0.0)
    out[...] = jnp.dot(h, w[...], preferred_element_type=jnp.float32) * dinv[...]


def _t2(q, y, dinv, b, w):
    grid = (N // BM,)
    return pl.pallas_call(
        _t2_body,
        grid=grid,
        in_specs=[
            pl.BlockSpec((1, BM, D), lambda i: (0, i, 0)),
            pl.BlockSpec((1, BM, D), lambda i: (1, i, 0)),
            pl.BlockSpec((BM, D), lambda i: (i, 0)),
            pl.BlockSpec((BM, 1), lambda i: (i, 0)),
            pl.BlockSpec((1, D), lambda i: (0, 0)),
            pl.BlockSpec((D, D), lambda i: (0, 0)),
        ],
        out_specs=pl.BlockSpec((BM, D), lambda i: (i, 0)),
        out_shape=jax.ShapeDtypeStruct((N, D), jnp.float32),
    )(q, q, y, dinv, b.reshape(1, D), w)


def _t3_body(q0, q1, y, dinv, b, fcw, fcb, out, acc):
    i = pl.program_id(0)

    @pl.when(i == 0)
    def _():
        acc[...] = jnp.zeros_like(acc)

    h = dinv[...] * (q0[0] + q1[0] + y[...]) + b[...]
    h = jnp.maximum(h, 0.0)
    acc[...] += jnp.sum(h, axis=0, keepdims=True)

    @pl.when(i == pl.num_programs(0) - 1)
    def _():
        g = acc[...] * (1.0 / N)
        logits = jnp.dot(g, fcw[...], preferred_element_type=jnp.float32) + fcb[...]
        m = jnp.max(logits, axis=1, keepdims=True)
        t = logits - m
        out[...] = t - jnp.log(jnp.sum(jnp.exp(t), axis=1, keepdims=True))


def _t3(q, y, dinv, b, fcw, fcb):
    grid = (N // BM,)
    return pl.pallas_call(
        _t3_body,
        grid=grid,
        in_specs=[
            pl.BlockSpec((1, BM, D), lambda i: (0, i, 0)),
            pl.BlockSpec((1, BM, D), lambda i: (1, i, 0)),
            pl.BlockSpec((BM, D), lambda i: (i, 0)),
            pl.BlockSpec((BM, 1), lambda i: (i, 0)),
            pl.BlockSpec((1, D), lambda i: (0, 0)),
            pl.BlockSpec((D, D), lambda i: (0, 0)),
            pl.BlockSpec((1, D), lambda i: (0, 0)),
        ],
        out_specs=pl.BlockSpec((1, D), lambda i: (0, 0)),
        out_shape=jax.ShapeDtypeStruct((1, D), jnp.float32),
        scratch_shapes=[pltpu.VMEM((1, D), jnp.float32)],
    )(q, q, y, dinv, b.reshape(1, D), fcw, fcb.reshape(1, D))


def kernel(x, edge_index, W1, b1, W2, b2, fc_W, fc_b):
    src = edge_index[0].astype(jnp.int32)
    dst = edge_index[1].astype(jnp.int32)
    e = src.shape[0]

    nch = -(-e // (NW * C))
    if nch % 2:
        nch += 1
    epad = NW * nch * C
    pad_n = epad - e
    if pad_n:
        pad_src = jnp.asarray((np.arange(pad_n) * 131) % N, jnp.int32)
        pad_dst = jnp.asarray(N + (np.arange(pad_n) % (NPAD - N)), jnp.int32)
        src = jnp.concatenate([src, pad_src])
        dst = jnp.concatenate([dst, pad_dst])
    srcp = src.reshape(NW, nch, 1, C)
    dstp = dst.reshape(NW, nch, 1, C)
    edges = jnp.concatenate([srcp, dstp], axis=2)
    dstp = dstp.reshape(NW, nch, C)

    zeros = jnp.zeros((NPAD, D), jnp.float32)
    zeros1 = jnp.zeros((NPAD, DW), jnp.float32)
    ones_c = jnp.ones((C2, DW), jnp.float32)

    nch2 = nch * C // C2
    deg_kernel = _make_deg(nch2)
    agg_kernel = _make_agg(nch)

    degp = deg_kernel(dstp.reshape(NW, nch2, C2), ones_c, zeros1)
    y1, dinv = _t1(degp, x, W1)
    q1 = agg_kernel(y1, edges, zeros)
    y2 = _t2(q1, y1, dinv, b1, W2)
    q2 = agg_kernel(y2, edges, zeros)
    return _t3(q2, y2, dinv, b2, fc_W, fc_b)
